# baseline (speedup 1.0000x reference)
import functools

import jax
import jax.numpy as jnp
from jax.experimental import pallas as pl
from jax.experimental.pallas import tpu as pltpu


def _scan_block(blk, rb):
    acc = blk
    s = 1
    while s < rb:
        shifted = jnp.concatenate(
            [jnp.zeros((s, acc.shape[1]), acc.dtype), acc[:-s]], axis=0
        )
        acc = acc + shifted
        s *= 2
    return acc


def _body(x_ref, o_ref, carry, *, rb):
    r = pl.program_id(2)

    @pl.when(r == 0)
    def _():
        carry[...] = jnp.zeros_like(carry)

    local = _scan_block(x_ref[0], rb)
    o_ref[0] = local + carry[...]
    carry[...] = carry[...] + local[rb - 1 : rb, :]


def kernel(x):
    b, n, c = x.shape
    rb = min(n, 64)
    cb = min(c, 4096)
    grid = (b, c // cb, n // rb)
    return pl.pallas_call(
        functools.partial(_body, rb=rb),
        grid=grid,
        in_specs=[
            pl.BlockSpec((1, rb, cb), lambda i, j, k: (i, k, j)),
        ],
        out_specs=pl.BlockSpec((1, rb, cb), lambda i, j, k: (i, k, j)),
        out_shape=jax.ShapeDtypeStruct((b, n, c), x.dtype),
        scratch_shapes=[pltpu.VMEM((1, cb), x.dtype)],
        compiler_params=pltpu.CompilerParams(
            dimension_semantics=("parallel", "parallel", "arbitrary"),
        ),
    )(x)


# speedup vs baseline: 1.5080x; 1.5080x over previous
import functools

import jax
import jax.numpy as jnp
from jax.experimental import pallas as pl
from jax.experimental.pallas import tpu as pltpu


def _scan_block(blk, rb):
    acc = blk
    s = 1
    while s < rb:
        shifted = jnp.concatenate(
            [jnp.zeros((s, acc.shape[1]), acc.dtype), acc[:-s]], axis=0
        )
        acc = acc + shifted
        s *= 2
    return acc


def _body(x_ref, o_ref, carry, *, rb):
    r = pl.program_id(2)

    @pl.when(r == 0)
    def _():
        carry[...] = jnp.zeros_like(carry)

    local = _scan_block(x_ref[0], rb)
    o_ref[0] = local + carry[...]
    carry[...] = carry[...] + local[rb - 1 : rb, :]


def kernel(x):
    b, n, c = x.shape
    rb = min(n, 512)
    cb = min(c, 4096)
    grid = (b, c // cb, n // rb)
    return pl.pallas_call(
        functools.partial(_body, rb=rb),
        grid=grid,
        in_specs=[
            pl.BlockSpec((1, rb, cb), lambda i, j, k: (i, k, j)),
        ],
        out_specs=pl.BlockSpec((1, rb, cb), lambda i, j, k: (i, k, j)),
        out_shape=jax.ShapeDtypeStruct((b, n, c), x.dtype),
        scratch_shapes=[pltpu.VMEM((1, cb), x.dtype)],
        compiler_params=pltpu.CompilerParams(
            dimension_semantics=("parallel", "parallel", "arbitrary"),
        ),
    )(x)
